# baseline (speedup 1.0000x reference)
import jax
import jax.numpy as jnp
from jax.experimental import pallas as pl

_N_NODES = 50000
_N_EDGES = 1600000
_N_UNARY = 8
_N_BINARY = 2

_U_ROWS = (_N_NODES * _N_UNARY) // 128
_B_ROWS = (_N_EDGES * _N_BINARY) // 128
_B_BLOCK = 1000


def _copy_both(u_ref, b_ref, ou_ref, ob_ref):
    i = pl.program_id(0)

    @pl.when(i == 0)
    def _():
        ou_ref[...] = u_ref[...]

    ob_ref[...] = b_ref[...]


def kernel(unary, binary, index1, index2):
    u2 = unary.reshape(_U_ROWS, 128)
    b2 = binary.reshape(_B_ROWS, 128)
    out_u, out_b = pl.pallas_call(
        _copy_both,
        grid=(_B_ROWS // _B_BLOCK,),
        in_specs=[
            pl.BlockSpec((_U_ROWS, 128), lambda i: (0, 0)),
            pl.BlockSpec((_B_BLOCK, 128), lambda i: (i, 0)),
        ],
        out_specs=[
            pl.BlockSpec((_U_ROWS, 128), lambda i: (0, 0)),
            pl.BlockSpec((_B_BLOCK, 128), lambda i: (i, 0)),
        ],
        out_shape=[
            jax.ShapeDtypeStruct((_U_ROWS, 128), jnp.float32),
            jax.ShapeDtypeStruct((_B_ROWS, 128), jnp.float32),
        ],
    )(u2, b2)
    return (out_u.reshape(_N_NODES, _N_UNARY),
            out_b.reshape(_N_EDGES, _N_BINARY))
